# Initial kernel scaffold; baseline (speedup 1.0000x reference)
import jax
import jax.numpy as jnp
from jax.experimental import pallas as pl


def kernel(positions, times, W1, b1, W2, b2, W3, b3, W4, b4):
    raise NotImplementedError("write your pallas kernel here")


# speedup vs baseline: 1.2095x; 1.2095x over previous
import functools

import jax
import jax.numpy as jnp
from jax import lax
from jax.experimental import pallas as pl
from jax.experimental.pallas import tpu as pltpu
from jax.experimental.pallas import tpu_sc as plsc

_C = 8
_W = 256
_B = 512
_NW = 32


def _route_body(t_ref, rank_ref, counts_ref):
    t = t_ref[...]
    rw = t.shape[1]
    rank = jnp.zeros_like(t)
    counts_cols = []
    for c in range(_C):
        oh = (t == c).astype(jnp.int32)
        pre = oh
        s = 1
        while s < rw:
            pre = pre + jnp.concatenate(
                [jnp.zeros((8, s), jnp.int32), pre[:, : rw - s]], axis=1)
            s *= 2
        row_tot = pre[:, rw - 1 : rw]
        inc = row_tot
        s = 1
        while s < 8:
            inc = inc + jnp.concatenate(
                [jnp.zeros((s, 1), jnp.int32), inc[: 8 - s, :]], axis=0)
            s *= 2
        row_off = inc - row_tot
        pre_full = pre + row_off
        rank = rank + jnp.where(oh == 1, pre_full - 1, 0)
        counts_cols.append(inc[7:8, :])
    rank_ref[...] = rank
    counts_ref[...] = jnp.concatenate(counts_cols, axis=0)


def _route(times2d):
    return pl.pallas_call(
        _route_body,
        out_shape=(
            jax.ShapeDtypeStruct(times2d.shape, jnp.int32),
            jax.ShapeDtypeStruct((_C, 1), jnp.int32),
        ),
    )(times2d)


def _dispatch_body(pos_hbm, times_hbm, rank_hbm, ps_hbm,
                   xpad_hbm, slot_hbm,
                   t_v, r_v, ps_v, idx2_v, pos_v, sem):
    ch = t_v.shape[0]
    nrow = ch // 128
    wid = lax.axis_index("s") * 2 + lax.axis_index("c")
    base = wid * ch
    pltpu.sync_copy(times_hbm.at[pl.ds(base, ch)], t_v)
    pltpu.sync_copy(rank_hbm.at[pl.ds(base, ch)], r_v)
    pltpu.sync_copy(ps_hbm, ps_v)
    pltpu.sync_copy(pos_hbm.at[pl.ds(base, ch)], pos_v)

    for row in range(nrow):
        def body(k, carry, row=row):
            off = row * 128 + k * 16
            t = t_v[pl.ds(off, 16)]
            r = r_v[pl.ds(off, 16)]
            ps = plsc.load_gather(ps_v, [t])
            idx2_v[row, pl.ds(k * 16, 16)] = ps + r
            return carry
        lax.fori_loop(0, 8, body, 0)

    for row in range(nrow):
        pltpu.sync_copy(idx2_v.at[row],
                        slot_hbm.at[pl.ds(base + row * 128, 128)])
    cps = [
        pltpu.async_copy(pos_v.at[pl.ds(row * 128, 128)],
                         xpad_hbm.at[idx2_v.at[row]], sem)
        for row in range(nrow)
    ]
    for cp in cps:
        cp.wait()


def _dispatch(pos8, times, rank, ps16, npad):
    n = pos8.shape[0]
    ch = n // _NW
    mesh = plsc.VectorSubcoreMesh(core_axis_name="c", subcore_axis_name="s")
    return pl.kernel(
        _dispatch_body,
        out_type=(
            jax.ShapeDtypeStruct((npad, 8), jnp.float32),
            jax.ShapeDtypeStruct((n,), jnp.int32),
        ),
        mesh=mesh,
        scratch_types=[
            pltpu.VMEM((ch,), jnp.int32),
            pltpu.VMEM((ch,), jnp.int32),
            pltpu.VMEM((16,), jnp.int32),
            pltpu.VMEM((ch // 128, 128), jnp.int32),
            pltpu.VMEM((ch, 8), jnp.float32),
            pltpu.SemaphoreType.DMA,
        ],
        compiler_params=pltpu.CompilerParams(needs_layout_passes=False,
                                             use_tc_tiling_on_sc=False),
    )(pos8, times, rank, ps16)


def _mlp_body(e_ref, x_ref, w1_ref, b1_ref, w2_ref, b2_ref,
              w3_ref, b3_ref, w4_ref, b4_ref, y_ref):
    x = x_ref[...]
    h = jnp.tanh(jnp.dot(x, w1_ref[0], preferred_element_type=jnp.float32)
                 + b1_ref[0])
    h = jnp.tanh(jnp.dot(h, w2_ref[0], preferred_element_type=jnp.float32)
                 + b2_ref[0])
    h = jnp.tanh(jnp.dot(h, w3_ref[0], preferred_element_type=jnp.float32)
                 + b3_ref[0])
    y_ref[...] = jnp.tanh(jnp.dot(h, w4_ref[0],
                                  preferred_element_type=jnp.float32)
                          + b4_ref[0])


def _mlp(block_expert, xpad, w1p, b1, w2, b2, w3, b3, w4p, b4p):
    npad = xpad.shape[0]
    nb = npad // _B
    grid_spec = pltpu.PrefetchScalarGridSpec(
        num_scalar_prefetch=1,
        grid=(nb,),
        in_specs=[
            pl.BlockSpec((_B, 8), lambda i, e: (i, 0)),
            pl.BlockSpec((1, 8, _W), lambda i, e: (e[i], 0, 0)),
            pl.BlockSpec((1, 1, _W), lambda i, e: (e[i], 0, 0)),
            pl.BlockSpec((1, _W, _W), lambda i, e: (e[i], 0, 0)),
            pl.BlockSpec((1, 1, _W), lambda i, e: (e[i], 0, 0)),
            pl.BlockSpec((1, _W, _W), lambda i, e: (e[i], 0, 0)),
            pl.BlockSpec((1, 1, _W), lambda i, e: (e[i], 0, 0)),
            pl.BlockSpec((1, _W, 8), lambda i, e: (e[i], 0, 0)),
            pl.BlockSpec((1, 1, 8), lambda i, e: (e[i], 0, 0)),
        ],
        out_specs=pl.BlockSpec((_B, 8), lambda i, e: (i, 0)),
    )
    return pl.pallas_call(
        _mlp_body,
        grid_spec=grid_spec,
        out_shape=jax.ShapeDtypeStruct((npad, 8), jnp.float32),
    )(block_expert, xpad, w1p,
      b1.reshape(_C, 1, _W), w2, b2.reshape(_C, 1, _W),
      w3, b3.reshape(_C, 1, _W), w4p, b4p.reshape(_C, 1, 8))


def _combine_body(ypad_hbm, slot_hbm, out_hbm, idx1_v, y_v, sem):
    ch = idx1_v.shape[0]
    wid = lax.axis_index("s") * 2 + lax.axis_index("c")
    base = wid * ch
    pltpu.sync_copy(slot_hbm.at[pl.ds(base, ch)], idx1_v)
    nrow = ch // 128
    cps = [
        pltpu.async_copy(ypad_hbm.at[idx1_v.at[pl.ds(row * 128, 128)]],
                         y_v.at[pl.ds(row * 128, 128)], sem)
        for row in range(nrow)
    ]
    for cp in cps:
        cp.wait()
    pltpu.sync_copy(y_v, out_hbm.at[pl.ds(base, ch)])


def _combine(ypad, slot):
    n = slot.shape[0]
    ch = n // _NW
    mesh = plsc.VectorSubcoreMesh(core_axis_name="c", subcore_axis_name="s")
    return pl.kernel(
        _combine_body,
        out_type=jax.ShapeDtypeStruct((n, 8), jnp.float32),
        mesh=mesh,
        scratch_types=[
            pltpu.VMEM((ch,), jnp.int32),
            pltpu.VMEM((ch, 8), jnp.float32),
            pltpu.SemaphoreType.DMA,
        ],
        compiler_params=pltpu.CompilerParams(needs_layout_passes=False,
                                             use_tc_tiling_on_sc=False),
    )(ypad, slot)


def kernel(positions, times, W1, b1, W2, b2, W3, b3, W4, b4):
    n = positions.shape[0]
    npad = n + _C * _B

    rank2d, counts2d = _route(times.astype(jnp.int32).reshape(_C, n // _C))
    rank = rank2d.reshape(n)
    counts = counts2d[:, 0]

    nblk = (counts + _B - 1) // _B
    cumblk = jnp.cumsum(nblk)
    padded_start = ((cumblk - nblk) * _B).astype(jnp.int32)
    ps16 = jnp.concatenate(
        [padded_start, jnp.zeros((8,), jnp.int32)])
    nb = npad // _B
    bids = jnp.arange(nb, dtype=jnp.int32)
    block_expert = jnp.minimum(
        jnp.sum((bids[:, None] >= cumblk[None, :].astype(jnp.int32))
                .astype(jnp.int32), axis=1),
        _C - 1).astype(jnp.int32)

    pos8 = jnp.pad(positions, ((0, 0), (0, 5)))
    xpad, slot = _dispatch(pos8, times.astype(jnp.int32), rank, ps16, npad)

    w1p = jnp.pad(W1, ((0, 0), (0, 5), (0, 0)))
    w4p = jnp.pad(W4, ((0, 0), (0, 0), (0, 5)))
    b4p = jnp.pad(b4, ((0, 0), (0, 5)))
    ypad = _mlp(block_expert, xpad, w1p, b1, W2, b2, W3, b3, w4p, b4p)

    out8 = _combine(ypad, slot)
    return out8[:, :3]
